# baseline (speedup 1.0000x reference)
import functools

import jax
import jax.numpy as jnp
from jax import lax
from jax.experimental import pallas as pl
from jax.experimental.pallas import tpu as pltpu
from jax.experimental.pallas import tpu_sc as plsc

NC = 2
NS = 16
NW = NC * NS
LANES = 16
CHUNK = 128
NBUF = 3
SLAB = 64


def _make_hist_kernel(NT, NCHUNK):
  mesh = plsc.VectorSubcoreMesh(core_axis_name="c", subcore_axis_name="s")
  rpt = NT // NS

  @functools.partial(
      pl.kernel,
      out_type=jax.ShapeDtypeStruct((NC, NT, LANES), jnp.float32),
      mesh=mesh,
      compiler_params=pltpu.CompilerParams(use_tc_tiling_on_sc=False),
      scratch_types=[
          pltpu.VMEM((NCHUNK, CHUNK), jnp.int32),
          pltpu.VMEM((CHUNK, LANES), jnp.float32),
          pltpu.VMEM_SHARED((NT, LANES), jnp.float32),
      ],
  )
  def hist_kernel(dst_hbm, ones_hbm, zeros_hbm, out_hbm, idx_v, ones_v,
                  hist_sh):
    c = lax.axis_index("c")
    s = lax.axis_index("s")
    wid = c * NS + s
    base = s * rpt
    pltpu.sync_copy(zeros_hbm.at[pl.ds(base, rpt)], hist_sh.at[pl.ds(base, rpt)])
    pltpu.sync_copy(ones_hbm, ones_v)
    pltpu.sync_copy(dst_hbm.at[wid], idx_v)
    plsc.subcore_barrier()

    def body(j, carry):
      pltpu.sync_copy(ones_v, hist_sh.at[idx_v.at[j]], add=True)
      return carry

    lax.fori_loop(0, NCHUNK, body, 0)
    plsc.subcore_barrier()
    pltpu.sync_copy(hist_sh.at[pl.ds(base, rpt)],
                    out_hbm.at[c, pl.ds(base, rpt)])

  return hist_kernel


def _make_agg_kernel(NT, NCHUNK, n_slabs):
  mesh = plsc.VectorSubcoreMesh(core_axis_name="c", subcore_axis_name="s")
  rpt = NT // NS
  assert NCHUNK % NBUF == 0
  niter = NCHUNK // NBUF

  @functools.partial(
      pl.kernel,
      out_type=jax.ShapeDtypeStruct((n_slabs, NC, NT, SLAB), jnp.float32),
      mesh=mesh,
      compiler_params=pltpu.CompilerParams(use_tc_tiling_on_sc=False),
      scratch_types=[
          pltpu.VMEM((NCHUNK, CHUNK), jnp.int32),
          pltpu.VMEM((NCHUNK, CHUNK), jnp.int32),
          [pltpu.VMEM((CHUNK, SLAB), jnp.float32) for _ in range(NBUF)],
          pltpu.VMEM_SHARED((NT, SLAB), jnp.float32),
          pltpu.VMEM_SHARED((NT, SLAB), jnp.float32),
          [pltpu.SemaphoreType.DMA for _ in range(NBUF)],
      ],
  )
  def agg_kernel(*refs):
    hs_hbms = refs[:n_slabs]
    src_hbm, dst_hbm, zeros_hbm, out_hbm = refs[n_slabs:n_slabs + 4]
    sidx, didx, rows, agg_sh, hs_sh, sems = refs[n_slabs + 4:]
    c = lax.axis_index("c")
    s = lax.axis_index("s")
    wid = c * NS + s
    base = s * rpt
    pltpu.sync_copy(src_hbm.at[wid], sidx)
    pltpu.sync_copy(dst_hbm.at[wid], didx)

    for sl in range(n_slabs):
      pltpu.sync_copy(hs_hbms[sl].at[pl.ds(base, rpt)],
                      hs_sh.at[pl.ds(base, rpt)])
      pltpu.sync_copy(zeros_hbm.at[pl.ds(base, rpt)],
                      agg_sh.at[pl.ds(base, rpt)])
      plsc.subcore_barrier()

      for b in range(NBUF):
        pltpu.async_copy(hs_sh.at[sidx.at[b]], rows[b], sems[b])

      def body(j, carry):
        ch0 = j * NBUF
        for b in range(NBUF):
          pltpu.make_async_copy(hs_sh.at[sidx.at[ch0 + b]], rows[b],
                                sems[b]).wait()
          pltpu.sync_copy(rows[b], agg_sh.at[didx.at[ch0 + b]], add=True)
          pltpu.async_copy(hs_sh.at[sidx.at[ch0 + b + NBUF]], rows[b],
                           sems[b])
        return carry

      lax.fori_loop(0, niter - 1, body, 0)
      for b in range(NBUF):
        ch = (niter - 1) * NBUF + b
        pltpu.make_async_copy(hs_sh.at[sidx.at[ch]], rows[b], sems[b]).wait()
        pltpu.sync_copy(rows[b], agg_sh.at[didx.at[ch]], add=True)

      plsc.subcore_barrier()
      pltpu.sync_copy(agg_sh.at[pl.ds(base, rpt)],
                      out_hbm.at[sl, c, pl.ds(base, rpt)])

  return agg_kernel


def _dinv_from_hist(hist_ref):
  deg = hist_ref[0, :, 0:1] + hist_ref[1, :, 0:1] + 1.0
  return lax.rsqrt(deg)


def _make_matmul_scale_body(n_slabs):
  def body(x_ref, w_ref, hist_ref, *out_refs):
    dinv = _dinv_from_hist(hist_ref)
    h = jnp.dot(x_ref[...], w_ref[...], preferred_element_type=jnp.float32)
    hs = h * dinv
    for i in range(n_slabs):
      out_refs[i][...] = hs[:, i * SLAB:(i + 1) * SLAB]
  return body


def _make_mid_layer_body(n_in, n_out):
  def body(*refs):
    agg_ref = refs[0]
    hss = refs[1:1 + n_in]
    hist_ref, w_ref, b_ref = refs[1 + n_in:4 + n_in]
    out_refs = refs[4 + n_in:]
    dinv = _dinv_from_hist(hist_ref)
    t = jnp.concatenate(
        [agg_ref[i, 0] + agg_ref[i, 1] + hss[i][...] for i in range(n_in)],
        axis=-1)
    h = jnp.maximum(t * dinv + b_ref[...], 0.0)
    hs = jnp.dot(h, w_ref[...], preferred_element_type=jnp.float32) * dinv
    for i in range(n_out):
      out_refs[i][...] = hs[:, i * SLAB:(i + 1) * SLAB]
  return body


def _make_final_body(n_in):
  def body(*refs):
    agg_ref = refs[0]
    hss = refs[1:1 + n_in]
    hist_ref, b_ref = refs[1 + n_in:3 + n_in]
    out_ref = refs[3 + n_in]
    dinv = _dinv_from_hist(hist_ref)
    t = jnp.concatenate(
        [agg_ref[i, 0] + agg_ref[i, 1] + hss[i][...] for i in range(n_in)],
        axis=-1)
    out_ref[...] = t * dinv + b_ref[...]
  return body


def kernel(x, edge_index, W1, b1, W2, b2):
  N, DIN = x.shape
  DH = W1.shape[1]
  DOUT = W2.shape[1]
  E = edge_index.shape[1]
  assert DH % SLAB == 0 and DOUT % SLAB == 0
  nsl_h = DH // SLAB
  nsl_o = DOUT // SLAB

  x = x.astype(jnp.float32)
  src = edge_index[0].astype(jnp.int32)
  dst = edge_index[1].astype(jnp.int32)

  NT = -(-(N + 1) // (NS * 8)) * (NS * 8)
  NCHUNK = -(-E // (NW * CHUNK))
  NCHUNK = -(-NCHUNK // NBUF) * NBUF
  E_pad = NW * NCHUNK * CHUNK
  src_p = jnp.concatenate([src, jnp.zeros((E_pad - E,), jnp.int32)])
  dst_p = jnp.concatenate([dst, jnp.full((E_pad - E,), N, jnp.int32)])
  src3 = src_p.reshape(NW, NCHUNK, CHUNK)
  dst3 = dst_p.reshape(NW, NCHUNK, CHUNK)

  ones_l = jnp.ones((CHUNK, LANES), jnp.float32)
  zeros_l = jnp.zeros((NT, LANES), jnp.float32)
  zeros_s = jnp.zeros((NT, SLAB), jnp.float32)

  hist = _make_hist_kernel(NT, NCHUNK)(dst3, ones_l, zeros_l)

  bm = 1024
  grid = (pl.cdiv(N, bm),)
  hist_spec = pl.BlockSpec((NC, bm, LANES), lambda m: (0, m, 0))
  slab_spec = pl.BlockSpec((bm, SLAB), lambda m: (m, 0))
  agg_spec_h = pl.BlockSpec((nsl_h, NC, bm, SLAB), lambda m: (0, 0, m, 0))
  agg_spec_o = pl.BlockSpec((nsl_o, NC, bm, SLAB), lambda m: (0, 0, m, 0))
  slab_shape = jax.ShapeDtypeStruct((NT, SLAB), jnp.float32)

  hs1 = pl.pallas_call(
      _make_matmul_scale_body(nsl_h),
      grid=grid,
      in_specs=[
          pl.BlockSpec((bm, DIN), lambda m: (m, 0)),
          pl.BlockSpec((DIN, DH), lambda m: (0, 0)),
          hist_spec,
      ],
      out_specs=[slab_spec] * nsl_h,
      out_shape=[slab_shape] * nsl_h,
  )(x, W1, hist)

  agg1 = _make_agg_kernel(NT, NCHUNK, nsl_h)(*hs1, src3, dst3, zeros_s)

  hs2 = pl.pallas_call(
      _make_mid_layer_body(nsl_h, nsl_o),
      grid=grid,
      in_specs=[agg_spec_h] + [slab_spec] * nsl_h + [
          hist_spec,
          pl.BlockSpec((DH, DOUT), lambda m: (0, 0)),
          pl.BlockSpec((1, DH), lambda m: (0, 0)),
      ],
      out_specs=[slab_spec] * nsl_o,
      out_shape=[slab_shape] * nsl_o,
  )(agg1, *hs1, hist, W2, b1.reshape(1, DH))

  agg2 = _make_agg_kernel(NT, NCHUNK, nsl_o)(*hs2, src3, dst3, zeros_s)

  out = pl.pallas_call(
      _make_final_body(nsl_o),
      grid=grid,
      in_specs=[agg_spec_o] + [slab_spec] * nsl_o + [
          hist_spec,
          pl.BlockSpec((1, DOUT), lambda m: (0, 0)),
      ],
      out_specs=pl.BlockSpec((bm, DOUT), lambda m: (m, 0)),
      out_shape=jax.ShapeDtypeStruct((N, DOUT), jnp.float32),
  )(agg2, *hs2, hist, b2.reshape(1, DOUT))

  return out


# speedup vs baseline: 1.0046x; 1.0046x over previous
import functools

import jax
import jax.numpy as jnp
from jax import lax
from jax.experimental import pallas as pl
from jax.experimental.pallas import tpu as pltpu
from jax.experimental.pallas import tpu_sc as plsc

NC = 2
NS = 16
NW = NC * NS
LANES = 16
CHUNK = 128
NBUF = 3
SLAB = 64


def _make_hist_kernel(NT, NCHUNK):
  mesh = plsc.VectorSubcoreMesh(core_axis_name="c", subcore_axis_name="s")
  rpt = NT // NS

  @functools.partial(
      pl.kernel,
      out_type=jax.ShapeDtypeStruct((NC, NT, LANES), jnp.float32),
      mesh=mesh,
      compiler_params=pltpu.CompilerParams(use_tc_tiling_on_sc=False),
      scratch_types=[
          pltpu.VMEM((NCHUNK, CHUNK), jnp.int32),
          pltpu.VMEM((CHUNK, LANES), jnp.float32),
          pltpu.VMEM_SHARED((NT, LANES), jnp.float32),
      ],
  )
  def hist_kernel(dst_hbm, ones_hbm, zeros_hbm, out_hbm, idx_v, ones_v,
                  hist_sh):
    c = lax.axis_index("c")
    s = lax.axis_index("s")
    wid = c * NS + s
    base = s * rpt
    pltpu.sync_copy(zeros_hbm.at[pl.ds(base, rpt)], hist_sh.at[pl.ds(base, rpt)])
    pltpu.sync_copy(ones_hbm, ones_v)
    pltpu.sync_copy(dst_hbm.at[wid], idx_v)
    plsc.subcore_barrier()

    def body(j, carry):
      pltpu.sync_copy(ones_v, hist_sh.at[idx_v.at[j]], add=True)
      return carry

    lax.fori_loop(0, NCHUNK, body, 0)
    plsc.subcore_barrier()
    pltpu.sync_copy(hist_sh.at[pl.ds(base, rpt)],
                    out_hbm.at[c, pl.ds(base, rpt)])

  return hist_kernel


def _make_agg_kernel(NT, NCHUNK, n_slabs):
  mesh = plsc.VectorSubcoreMesh(core_axis_name="c", subcore_axis_name="s")
  rpt = NT // NS
  assert NCHUNK % NBUF == 0
  niter = NCHUNK // NBUF

  @functools.partial(
      pl.kernel,
      out_type=jax.ShapeDtypeStruct((n_slabs, NC, NT, SLAB), jnp.float32),
      mesh=mesh,
      compiler_params=pltpu.CompilerParams(use_tc_tiling_on_sc=False),
      scratch_types=[
          pltpu.VMEM((NCHUNK, CHUNK), jnp.int32),
          pltpu.VMEM((NCHUNK, CHUNK), jnp.int32),
          [pltpu.VMEM((CHUNK, SLAB), jnp.float32) for _ in range(NBUF)],
          pltpu.VMEM_SHARED((NT, SLAB), jnp.float32),
          pltpu.VMEM_SHARED((NT, SLAB), jnp.float32),
          [pltpu.SemaphoreType.DMA for _ in range(NBUF)],
      ],
  )
  def agg_kernel(*refs):
    hs_hbms = refs[:n_slabs]
    src_hbm, dst_hbm, zeros_hbm, out_hbm = refs[n_slabs:n_slabs + 4]
    sidx, didx, rows, agg_sh, hs_sh, sems = refs[n_slabs + 4:]
    c = lax.axis_index("c")
    s = lax.axis_index("s")
    wid = c * NS + s
    base = s * rpt
    pltpu.sync_copy(src_hbm.at[wid], sidx)
    pltpu.sync_copy(dst_hbm.at[wid], didx)

    for sl in range(n_slabs):
      pltpu.sync_copy(hs_hbms[sl].at[pl.ds(base, rpt)],
                      hs_sh.at[pl.ds(base, rpt)])
      pltpu.sync_copy(zeros_hbm.at[pl.ds(base, rpt)],
                      agg_sh.at[pl.ds(base, rpt)])
      plsc.subcore_barrier()

      for b in range(NBUF):
        pltpu.async_copy(hs_sh.at[sidx.at[b]], rows[b], sems[b])

      def body(j, carry):
        ch0 = j * NBUF
        for b in range(NBUF):
          pltpu.make_async_copy(hs_sh.at[sidx.at[ch0 + b]], rows[b],
                                sems[b]).wait()
          pltpu.sync_copy(rows[b], agg_sh.at[didx.at[ch0 + b]], add=True)
          pltpu.async_copy(hs_sh.at[sidx.at[ch0 + b + NBUF]], rows[b],
                           sems[b])
        return carry

      lax.fori_loop(0, niter - 1, body, 0)
      for b in range(NBUF):
        ch = (niter - 1) * NBUF + b
        pltpu.make_async_copy(hs_sh.at[sidx.at[ch]], rows[b], sems[b]).wait()
        pltpu.sync_copy(rows[b], agg_sh.at[didx.at[ch]], add=True)

      plsc.subcore_barrier()
      pltpu.sync_copy(agg_sh.at[pl.ds(base, rpt)],
                      out_hbm.at[sl, c, pl.ds(base, rpt)])

  return agg_kernel


def _dinv_from_hist(hist_ref):
  deg = hist_ref[0, :, 0:1] + hist_ref[1, :, 0:1] + 1.0
  return lax.rsqrt(deg)


def _make_matmul_body(n_slabs):
  def body(x_ref, w_ref, *out_refs):
    h = jnp.dot(x_ref[...], w_ref[...], preferred_element_type=jnp.float32)
    for i in range(n_slabs):
      out_refs[i][...] = h[:, i * SLAB:(i + 1) * SLAB]
  return body


def _make_scale_body(n_slabs):
  def body(*refs):
    hs_in = refs[:n_slabs]
    hist_ref = refs[n_slabs]
    out_refs = refs[n_slabs + 1:]
    dinv = _dinv_from_hist(hist_ref)
    for i in range(n_slabs):
      out_refs[i][...] = hs_in[i][...] * dinv
  return body


def _make_mid_layer_body(n_in, n_out):
  def body(*refs):
    agg_ref = refs[0]
    hss = refs[1:1 + n_in]
    hist_ref, w_ref, b_ref = refs[1 + n_in:4 + n_in]
    out_refs = refs[4 + n_in:]
    dinv = _dinv_from_hist(hist_ref)
    t = jnp.concatenate(
        [agg_ref[i, 0] + agg_ref[i, 1] + hss[i][...] for i in range(n_in)],
        axis=-1)
    h = jnp.maximum(t * dinv + b_ref[...], 0.0)
    hs = jnp.dot(h, w_ref[...], preferred_element_type=jnp.float32) * dinv
    for i in range(n_out):
      out_refs[i][...] = hs[:, i * SLAB:(i + 1) * SLAB]
  return body


def _make_final_body(n_in):
  def body(*refs):
    agg_ref = refs[0]
    hss = refs[1:1 + n_in]
    hist_ref, b_ref = refs[1 + n_in:3 + n_in]
    out_ref = refs[3 + n_in]
    dinv = _dinv_from_hist(hist_ref)
    t = jnp.concatenate(
        [agg_ref[i, 0] + agg_ref[i, 1] + hss[i][...] for i in range(n_in)],
        axis=-1)
    out_ref[...] = t * dinv + b_ref[...]
  return body


def kernel(x, edge_index, W1, b1, W2, b2):
  N, DIN = x.shape
  DH = W1.shape[1]
  DOUT = W2.shape[1]
  E = edge_index.shape[1]
  assert DH % SLAB == 0 and DOUT % SLAB == 0
  nsl_h = DH // SLAB
  nsl_o = DOUT // SLAB

  x = x.astype(jnp.float32)
  src = edge_index[0].astype(jnp.int32)
  dst = edge_index[1].astype(jnp.int32)

  NT = -(-(N + 1) // (NS * 8)) * (NS * 8)
  NCHUNK = -(-E // (NW * CHUNK))
  NCHUNK = -(-NCHUNK // NBUF) * NBUF
  E_pad = NW * NCHUNK * CHUNK
  src_p = jnp.concatenate([src, jnp.zeros((E_pad - E,), jnp.int32)])
  dst_p = jnp.concatenate([dst, jnp.full((E_pad - E,), N, jnp.int32)])
  src3 = src_p.reshape(NW, NCHUNK, CHUNK)
  dst3 = dst_p.reshape(NW, NCHUNK, CHUNK)

  ones_l = jnp.ones((CHUNK, LANES), jnp.float32)
  zeros_l = jnp.zeros((NT, LANES), jnp.float32)
  zeros_s = jnp.zeros((NT, SLAB), jnp.float32)

  hist = _make_hist_kernel(NT, NCHUNK)(dst3, ones_l, zeros_l)

  bm = 2048
  grid = (pl.cdiv(N, bm),)
  hist_spec = pl.BlockSpec((NC, bm, LANES), lambda m: (0, m, 0))
  slab_spec = pl.BlockSpec((bm, SLAB), lambda m: (m, 0))
  agg_spec_h = pl.BlockSpec((nsl_h, NC, bm, SLAB), lambda m: (0, 0, m, 0))
  agg_spec_o = pl.BlockSpec((nsl_o, NC, bm, SLAB), lambda m: (0, 0, m, 0))
  slab_shape = jax.ShapeDtypeStruct((NT, SLAB), jnp.float32)

  h1 = pl.pallas_call(
      _make_matmul_body(nsl_h),
      grid=grid,
      in_specs=[
          pl.BlockSpec((bm, DIN), lambda m: (m, 0)),
          pl.BlockSpec((DIN, DH), lambda m: (0, 0)),
      ],
      out_specs=[slab_spec] * nsl_h,
      out_shape=[slab_shape] * nsl_h,
  )(x, W1)

  hs1 = pl.pallas_call(
      _make_scale_body(nsl_h),
      grid=grid,
      in_specs=[slab_spec] * nsl_h + [hist_spec],
      out_specs=[slab_spec] * nsl_h,
      out_shape=[slab_shape] * nsl_h,
  )(*h1, hist)

  agg1 = _make_agg_kernel(NT, NCHUNK, nsl_h)(*hs1, src3, dst3, zeros_s)

  hs2 = pl.pallas_call(
      _make_mid_layer_body(nsl_h, nsl_o),
      grid=grid,
      in_specs=[agg_spec_h] + [slab_spec] * nsl_h + [
          hist_spec,
          pl.BlockSpec((DH, DOUT), lambda m: (0, 0)),
          pl.BlockSpec((1, DH), lambda m: (0, 0)),
      ],
      out_specs=[slab_spec] * nsl_o,
      out_shape=[slab_shape] * nsl_o,
  )(agg1, *hs1, hist, W2, b1.reshape(1, DH))

  agg2 = _make_agg_kernel(NT, NCHUNK, nsl_o)(*hs2, src3, dst3, zeros_s)

  out = pl.pallas_call(
      _make_final_body(nsl_o),
      grid=grid,
      in_specs=[agg_spec_o] + [slab_spec] * nsl_o + [
          hist_spec,
          pl.BlockSpec((1, DOUT), lambda m: (0, 0)),
      ],
      out_specs=pl.BlockSpec((bm, DOUT), lambda m: (m, 0)),
      out_shape=jax.ShapeDtypeStruct((N, DOUT), jnp.float32),
  )(agg2, *hs2, hist, b2.reshape(1, DOUT))

  return out


# speedup vs baseline: 1.1023x; 1.0972x over previous
import functools

import jax
import jax.numpy as jnp
from jax import lax
from jax.experimental import pallas as pl
from jax.experimental.pallas import tpu as pltpu
from jax.experimental.pallas import tpu_sc as plsc

NC = 2
NS = 16
NW = NC * NS
LANES = 16
CHUNK = 128
ACHUNK = 112
NBUF = 4
SLAB = 64


def _make_hist_kernel(NT, NCHUNK):
  mesh = plsc.VectorSubcoreMesh(core_axis_name="c", subcore_axis_name="s")
  rpt = NT // NS

  @functools.partial(
      pl.kernel,
      out_type=jax.ShapeDtypeStruct((NC, NT, LANES), jnp.float32),
      mesh=mesh,
      compiler_params=pltpu.CompilerParams(use_tc_tiling_on_sc=False),
      scratch_types=[
          pltpu.VMEM((NCHUNK, CHUNK), jnp.int32),
          pltpu.VMEM((CHUNK, LANES), jnp.float32),
          pltpu.VMEM_SHARED((NT, LANES), jnp.float32),
      ],
  )
  def hist_kernel(dst_hbm, ones_hbm, zeros_hbm, out_hbm, idx_v, ones_v,
                  hist_sh):
    c = lax.axis_index("c")
    s = lax.axis_index("s")
    wid = c * NS + s
    base = s * rpt
    pltpu.sync_copy(zeros_hbm.at[pl.ds(base, rpt)], hist_sh.at[pl.ds(base, rpt)])
    pltpu.sync_copy(ones_hbm, ones_v)
    pltpu.sync_copy(dst_hbm.at[wid], idx_v)
    plsc.subcore_barrier()

    def body(j, carry):
      pltpu.sync_copy(ones_v, hist_sh.at[idx_v.at[j]], add=True)
      return carry

    lax.fori_loop(0, NCHUNK, body, 0)
    plsc.subcore_barrier()
    pltpu.sync_copy(hist_sh.at[pl.ds(base, rpt)],
                    out_hbm.at[c, pl.ds(base, rpt)])

  return hist_kernel


def _make_agg_kernel(NT, NCHUNK, n_slabs):
  mesh = plsc.VectorSubcoreMesh(core_axis_name="c", subcore_axis_name="s")
  rpt = NT // NS
  assert NCHUNK % 4 == 0 and NCHUNK >= 8

  @functools.partial(
      pl.kernel,
      out_type=jax.ShapeDtypeStruct((n_slabs, NC, NT, SLAB), jnp.float32),
      mesh=mesh,
      compiler_params=pltpu.CompilerParams(use_tc_tiling_on_sc=False),
      scratch_types=[
          pltpu.VMEM((NCHUNK, ACHUNK), jnp.int32),
          pltpu.VMEM((NCHUNK, ACHUNK), jnp.int32),
          [pltpu.VMEM((ACHUNK, SLAB), jnp.float32) for _ in range(NBUF)],
          pltpu.VMEM_SHARED((NT, SLAB), jnp.float32),
          pltpu.VMEM_SHARED((NT, SLAB), jnp.float32),
          [pltpu.SemaphoreType.DMA for _ in range(NBUF)],
          [pltpu.SemaphoreType.DMA for _ in range(NBUF)],
      ],
  )
  def agg_kernel(*refs):
    hs_hbms = refs[:n_slabs]
    src_hbm, dst_hbm, zeros_hbm, out_hbm = refs[n_slabs:n_slabs + 4]
    sidx, didx, rows, agg_sh, hs_sh, gsem, ssem = refs[n_slabs + 4:]
    c = lax.axis_index("c")
    s = lax.axis_index("s")
    wid = c * NS + s
    base = s * rpt
    pltpu.sync_copy(src_hbm.at[wid], sidx)
    pltpu.sync_copy(dst_hbm.at[wid], didx)

    def gfire(ch, b):
      pltpu.async_copy(hs_sh.at[sidx.at[ch]], rows[b], gsem[b])

    def gwait(ch, b):
      pltpu.make_async_copy(hs_sh.at[sidx.at[ch]], rows[b], gsem[b]).wait()

    def sfire(ch, b):
      pltpu.async_copy(rows[b], agg_sh.at[didx.at[ch]], ssem[b], add=True)

    def swait(ch, b):
      pltpu.make_async_copy(rows[b], agg_sh.at[didx.at[ch]], ssem[b]).wait()

    for sl in range(n_slabs):
      pltpu.sync_copy(hs_hbms[sl].at[pl.ds(base, rpt)],
                      hs_sh.at[pl.ds(base, rpt)])
      pltpu.sync_copy(zeros_hbm.at[pl.ds(base, rpt)],
                      agg_sh.at[pl.ds(base, rpt)])
      plsc.subcore_barrier()

      gfire(0, 0)
      gfire(1, 1)
      for ch in (0, 1):
        gwait(ch, ch)
        sfire(ch, ch)
        gfire(ch + 2, ch + 2)

      def body(j, carry):
        base_ch = 2 + j * 4
        for k in range(4):
          ch = base_ch + k
          b = (2 + k) % 4
          bg = (b + 2) % 4
          gwait(ch, b)
          sfire(ch, b)
          swait(ch - 2, bg)
          gfire(ch + 2, bg)
        return carry

      lax.fori_loop(0, (NCHUNK - 4) // 4, body, 0)
      for ch in (NCHUNK - 2, NCHUNK - 1):
        b = ch % 4
        gwait(ch, b)
        sfire(ch, b)
        swait(ch - 2, (ch - 2) % 4)
      for ch in (NCHUNK - 2, NCHUNK - 1):
        swait(ch, ch % 4)

      plsc.subcore_barrier()
      pltpu.sync_copy(agg_sh.at[pl.ds(base, rpt)],
                      out_hbm.at[sl, c, pl.ds(base, rpt)])

  return agg_kernel


def _dinv_from_hist(hist_ref):
  deg = hist_ref[0, :, 0:1] + hist_ref[1, :, 0:1] + 1.0
  return lax.rsqrt(deg)


def _make_matmul_body(n_slabs):
  def body(x_ref, w_ref, *out_refs):
    h = jnp.dot(x_ref[...], w_ref[...], preferred_element_type=jnp.float32)
    for i in range(n_slabs):
      out_refs[i][...] = h[:, i * SLAB:(i + 1) * SLAB]
  return body


def _make_scale_body(n_slabs):
  def body(*refs):
    hs_in = refs[:n_slabs]
    hist_ref = refs[n_slabs]
    out_refs = refs[n_slabs + 1:]
    dinv = _dinv_from_hist(hist_ref)
    for i in range(n_slabs):
      out_refs[i][...] = hs_in[i][...] * dinv
  return body


def _make_mid_layer_body(n_in, n_out):
  def body(*refs):
    agg_ref = refs[0]
    hss = refs[1:1 + n_in]
    hist_ref, w_ref, b_ref = refs[1 + n_in:4 + n_in]
    out_refs = refs[4 + n_in:]
    dinv = _dinv_from_hist(hist_ref)
    t = jnp.concatenate(
        [agg_ref[i, 0] + agg_ref[i, 1] + hss[i][...] for i in range(n_in)],
        axis=-1)
    h = jnp.maximum(t * dinv + b_ref[...], 0.0)
    hs = jnp.dot(h, w_ref[...], preferred_element_type=jnp.float32) * dinv
    for i in range(n_out):
      out_refs[i][...] = hs[:, i * SLAB:(i + 1) * SLAB]
  return body


def _make_final_body(n_in):
  def body(*refs):
    agg_ref = refs[0]
    hss = refs[1:1 + n_in]
    hist_ref, b_ref = refs[1 + n_in:3 + n_in]
    out_ref = refs[3 + n_in]
    dinv = _dinv_from_hist(hist_ref)
    t = jnp.concatenate(
        [agg_ref[i, 0] + agg_ref[i, 1] + hss[i][...] for i in range(n_in)],
        axis=-1)
    out_ref[...] = t * dinv + b_ref[...]
  return body


def kernel(x, edge_index, W1, b1, W2, b2):
  N, DIN = x.shape
  DH = W1.shape[1]
  DOUT = W2.shape[1]
  E = edge_index.shape[1]
  assert DH % SLAB == 0 and DOUT % SLAB == 0
  nsl_h = DH // SLAB
  nsl_o = DOUT // SLAB

  x = x.astype(jnp.float32)
  src = edge_index[0].astype(jnp.int32)
  dst = edge_index[1].astype(jnp.int32)

  NT = -(-(N + 1) // (NS * 8)) * (NS * 8)
  NCHUNK = -(-E // (NW * CHUNK))
  E_pad = NW * NCHUNK * CHUNK
  dst_p = jnp.concatenate([dst, jnp.full((E_pad - E,), N, jnp.int32)])
  dst3 = dst_p.reshape(NW, NCHUNK, CHUNK)
  ANCHUNK = max(8, -(-(-(-E // (NW * ACHUNK))) // 4) * 4)
  EA_pad = NW * ANCHUNK * ACHUNK
  src_a = jnp.concatenate([src, jnp.zeros((EA_pad - E,), jnp.int32)])
  dst_a = jnp.concatenate([dst, jnp.full((EA_pad - E,), N, jnp.int32)])
  src3a = src_a.reshape(NW, ANCHUNK, ACHUNK)
  dst3a = dst_a.reshape(NW, ANCHUNK, ACHUNK)

  ones_l = jnp.ones((CHUNK, LANES), jnp.float32)
  zeros_l = jnp.zeros((NT, LANES), jnp.float32)
  zeros_s = jnp.zeros((NT, SLAB), jnp.float32)

  hist = _make_hist_kernel(NT, NCHUNK)(dst3, ones_l, zeros_l)

  bm = 2048
  grid = (pl.cdiv(N, bm),)
  hist_spec = pl.BlockSpec((NC, bm, LANES), lambda m: (0, m, 0))
  slab_spec = pl.BlockSpec((bm, SLAB), lambda m: (m, 0))
  agg_spec_h = pl.BlockSpec((nsl_h, NC, bm, SLAB), lambda m: (0, 0, m, 0))
  agg_spec_o = pl.BlockSpec((nsl_o, NC, bm, SLAB), lambda m: (0, 0, m, 0))
  slab_shape = jax.ShapeDtypeStruct((NT, SLAB), jnp.float32)

  h1 = pl.pallas_call(
      _make_matmul_body(nsl_h),
      grid=grid,
      in_specs=[
          pl.BlockSpec((bm, DIN), lambda m: (m, 0)),
          pl.BlockSpec((DIN, DH), lambda m: (0, 0)),
      ],
      out_specs=[slab_spec] * nsl_h,
      out_shape=[slab_shape] * nsl_h,
  )(x, W1)

  hs1 = pl.pallas_call(
      _make_scale_body(nsl_h),
      grid=grid,
      in_specs=[slab_spec] * nsl_h + [hist_spec],
      out_specs=[slab_spec] * nsl_h,
      out_shape=[slab_shape] * nsl_h,
  )(*h1, hist)

  agg1 = _make_agg_kernel(NT, ANCHUNK, nsl_h)(*hs1, src3a, dst3a, zeros_s)

  hs2 = pl.pallas_call(
      _make_mid_layer_body(nsl_h, nsl_o),
      grid=grid,
      in_specs=[agg_spec_h] + [slab_spec] * nsl_h + [
          hist_spec,
          pl.BlockSpec((DH, DOUT), lambda m: (0, 0)),
          pl.BlockSpec((1, DH), lambda m: (0, 0)),
      ],
      out_specs=[slab_spec] * nsl_o,
      out_shape=[slab_shape] * nsl_o,
  )(agg1, *hs1, hist, W2, b1.reshape(1, DH))

  agg2 = _make_agg_kernel(NT, ANCHUNK, nsl_o)(*hs2, src3a, dst3a, zeros_s)

  out = pl.pallas_call(
      _make_final_body(nsl_o),
      grid=grid,
      in_specs=[agg_spec_o] + [slab_spec] * nsl_o + [
          hist_spec,
          pl.BlockSpec((1, DOUT), lambda m: (0, 0)),
      ],
      out_specs=pl.BlockSpec((bm, DOUT), lambda m: (m, 0)),
      out_shape=jax.ShapeDtypeStruct((N, DOUT), jnp.float32),
  )(agg2, *hs2, hist, b2.reshape(1, DOUT))

  return out
